# baseline (speedup 1.0000x reference)
import functools

import jax
import jax.numpy as jnp
from jax import lax
from jax.experimental import pallas as pl
from jax.experimental.pallas import tpu as pltpu
from jax.experimental.pallas import tpu_sc as plsc

NUM_E3 = 12
E3_DIM = 64
HID = 32
BATCH = 16384

_NC = 2
_NS = 16
_L = 16
_NW = _NC * _NS
_BPW = BATCH // _NW

_GDN = lax.GatherDimensionNumbers(
    offset_dims=(), collapsed_slice_dims=(0,), start_index_map=(0,))


def _take16(vec, idx):
    return lax.gather(vec, idx.reshape(_L, 1), _GDN, (1,),
                      mode=lax.GatherScatterMode.PROMISE_IN_BOUNDS)


@functools.lru_cache(maxsize=None)
def _fused_call():
    mesh = plsc.VectorSubcoreMesh(core_axis_name="c", subcore_axis_name="s")

    @functools.partial(
        pl.kernel,
        mesh=mesh,
        out_type=[
            jax.ShapeDtypeStruct((BATCH,), jnp.float32),
            jax.ShapeDtypeStruct((BATCH,), jnp.float32),
        ],
        scratch_types=[
            pltpu.VMEM((_BPW,), jnp.int32),
            pltpu.VMEM((NUM_E3, E3_DIM), jnp.float32),
            pltpu.VMEM((E3_DIM, HID), jnp.float32),
            pltpu.VMEM((HID,), jnp.float32),
            pltpu.VMEM((HID,), jnp.float32),
            pltpu.VMEM((_L,), jnp.float32),
            pltpu.VMEM((_BPW,), jnp.float32),
            pltpu.VMEM((_BPW,), jnp.float32),
            pltpu.SemaphoreType.DMA,
        ],
    )
    def fused(idx_hbm, tab_hbm, w1_hbm, b1_hbm, w2_hbm, b2_hbm,
              out_l_hbm, out_s_hbm,
              idx_v, tab_v, w1_v, b1_v, w2_v, b2_v, ol_v, os_v, sem):
        wid = lax.axis_index("s") * _NC + lax.axis_index("c")
        base = wid * _BPW
        idx_cp = pltpu.async_copy(idx_hbm.at[pl.ds(base, _BPW)], idx_v, sem)
        pltpu.sync_copy(tab_hbm, tab_v)
        pltpu.sync_copy(w1_hbm, w1_v)
        pltpu.sync_copy(b1_hbm, b1_v)
        pltpu.sync_copy(w2_hbm, w2_v)
        pltpu.sync_copy(b2_hbm, b2_v.at[pl.ds(0, 1)])

        b1c0 = b1_v[pl.ds(0, _L)]
        b1c1 = b1_v[pl.ds(_L, _L)]
        acc0 = [b1c0] * NUM_E3
        acc1 = [b1c1] * NUM_E3
        for kc in range(E3_DIM // _L):
            k0 = kc * _L
            tcs = [tab_v[r, pl.ds(k0, _L)] for r in range(NUM_E3)]
            for dk in range(_L):
                w0 = w1_v[k0 + dk, pl.ds(0, _L)]
                w1r = w1_v[k0 + dk, pl.ds(_L, _L)]
                for r in range(NUM_E3):
                    t = tcs[r][dk]
                    acc0[r] = acc0[r] + t * w0
                    acc1[r] = acc1[r] + t * w1r

        w2c0 = w2_v[pl.ds(0, _L)]
        w2c1 = w2_v[pl.ds(_L, _L)]
        lane = lax.iota(jnp.int32, _L)
        b2bc = _take16(b2_v[...], jnp.zeros((_L,), jnp.int32))
        tl = jnp.zeros((_L,), jnp.float32)
        for r in range(NUM_E3):
            h0 = jnp.maximum(acc0[r], 0.0)
            h1 = jnp.maximum(acc1[r], 0.0)
            s = h0 * w2c0 + h1 * w2c1
            for sh in (8, 4, 2, 1):
                s = s + _take16(s, jnp.bitwise_xor(lane, sh))
            tl = jnp.where(lane == r, s, tl)
        tl = tl + b2bc
        ts = 1.0 / (1.0 + jnp.exp(-tl))

        idx_cp.wait()
        for i in range(_BPW // _L):
            iv = idx_v[pl.ds(i * _L, _L)]
            ol_v[pl.ds(i * _L, _L)] = _take16(tl, iv)
            os_v[pl.ds(i * _L, _L)] = _take16(ts, iv)
        pltpu.sync_copy(ol_v, out_l_hbm.at[pl.ds(base, _BPW)])
        pltpu.sync_copy(os_v, out_s_hbm.at[pl.ds(base, _BPW)])

    return fused


def kernel(e3_idx, table, W1, b1, W2, b2):
    idx = e3_idx.astype(jnp.int32)
    logits, score = _fused_call()(idx, table, W1, b1, W2.reshape(HID), b2)
    return logits, score


# speedup vs baseline: 1.2533x; 1.2533x over previous
import functools

import jax
import jax.numpy as jnp
from jax import lax
from jax.experimental import pallas as pl
from jax.experimental.pallas import tpu as pltpu
from jax.experimental.pallas import tpu_sc as plsc

NUM_E3 = 12
E3_DIM = 64
HID = 32
BATCH = 16384

_NC = 2
_NS = 16
_L = 16
_NW = _NC * _NS
_BPW = BATCH // _NW

_GDN = lax.GatherDimensionNumbers(
    offset_dims=(), collapsed_slice_dims=(0,), start_index_map=(0,))


def _take16(vec, idx):
    return lax.gather(vec, idx.reshape(_L, 1), _GDN, (1,),
                      mode=lax.GatherScatterMode.PROMISE_IN_BOUNDS)


def _mlp_body(tab_ref, w1_ref, b1_ref, w2_ref, b2_ref, lg_ref, sc_ref):
    t = tab_ref[...]
    h = jnp.maximum(
        jnp.dot(t, w1_ref[...], preferred_element_type=jnp.float32)
        + b1_ref[...],
        0.0,
    )
    lg = lax.dot_general(w2_ref[...], h, (((1,), (1,)), ((), ())),
                         preferred_element_type=jnp.float32) + b2_ref[...]
    lg16 = jnp.pad(lg, ((0, 0), (0, _L - NUM_E3)))
    lg_ref[...] = lg16
    sc_ref[...] = jax.nn.sigmoid(lg16)


@functools.lru_cache(maxsize=None)
def _mlp_call():
    return pl.pallas_call(
        _mlp_body,
        out_shape=[
            jax.ShapeDtypeStruct((1, _L), jnp.float32),
            jax.ShapeDtypeStruct((1, _L), jnp.float32),
        ],
    )


@functools.lru_cache(maxsize=None)
def _gather_call():
    mesh = plsc.VectorSubcoreMesh(core_axis_name="c", subcore_axis_name="s")

    @functools.partial(
        pl.kernel,
        mesh=mesh,
        out_type=[
            jax.ShapeDtypeStruct((BATCH,), jnp.float32),
            jax.ShapeDtypeStruct((BATCH,), jnp.float32),
        ],
        scratch_types=[
            pltpu.VMEM((_BPW,), jnp.int32),
            pltpu.VMEM((_L,), jnp.float32),
            pltpu.VMEM((_L,), jnp.float32),
            pltpu.VMEM((_BPW,), jnp.float32),
            pltpu.VMEM((_BPW,), jnp.float32),
            pltpu.SemaphoreType.DMA,
            pltpu.SemaphoreType.DMA,
            pltpu.SemaphoreType.DMA,
        ],
    )
    def sc_gather(idx_hbm, tl_hbm, ts_hbm, out_l_hbm, out_s_hbm,
                  idx_v, tl_v, ts_v, ol_v, os_v, sem_i, sem_l, sem_s):
        wid = lax.axis_index("s") * _NC + lax.axis_index("c")
        base = wid * _BPW
        idx_cp = pltpu.async_copy(idx_hbm.at[pl.ds(base, _BPW)], idx_v, sem_i)
        pltpu.sync_copy(tl_hbm, tl_v)
        pltpu.sync_copy(ts_hbm, ts_v)
        tl = tl_v[...]
        ts = ts_v[...]
        idx_cp.wait()
        for i in range(_BPW // _L):
            iv = idx_v[pl.ds(i * _L, _L)]
            ol_v[pl.ds(i * _L, _L)] = _take16(tl, iv)
        l_cp = pltpu.async_copy(ol_v, out_l_hbm.at[pl.ds(base, _BPW)], sem_l)
        for i in range(_BPW // _L):
            iv = idx_v[pl.ds(i * _L, _L)]
            os_v[pl.ds(i * _L, _L)] = _take16(ts, iv)
        s_cp = pltpu.async_copy(os_v, out_s_hbm.at[pl.ds(base, _BPW)], sem_s)
        l_cp.wait()
        s_cp.wait()

    return sc_gather


def kernel(e3_idx, table, W1, b1, W2, b2):
    idx = e3_idx.astype(jnp.int32)
    tl, ts = _mlp_call()(table, W1, b1.reshape(1, HID), W2.reshape(1, HID),
                         b2.reshape(1, 1))
    logits, score = _gather_call()(idx, tl.reshape(_L), ts.reshape(_L))
    return logits, score


# speedup vs baseline: 1.2923x; 1.0311x over previous
import functools

import jax
import jax.numpy as jnp
from jax import lax
from jax.experimental import pallas as pl
from jax.experimental.pallas import tpu as pltpu
from jax.experimental.pallas import tpu_sc as plsc

NUM_E3 = 12
E3_DIM = 64
HID = 32
BATCH = 16384

_NC = 2
_NS = 16
_L = 16
_NW = _NC * _NS
_BPW = BATCH // _NW

_GDN = lax.GatherDimensionNumbers(
    offset_dims=(), collapsed_slice_dims=(0,), start_index_map=(0,))


def _take16(vec, idx):
    return lax.gather(vec, idx.reshape(_L, 1), _GDN, (1,),
                      mode=lax.GatherScatterMode.PROMISE_IN_BOUNDS)


def _mlp_body(tab_ref, w1_ref, b1_ref, w2_ref, b2_ref, out_ref):
    t = tab_ref[...]
    w1 = w1_ref[...]
    h = jnp.maximum(
        jnp.dot(t, w1, preferred_element_type=jnp.float32) + b1_ref[...],
        0.0,
    )
    lg = lax.dot_general(w2_ref[...], h, (((1,), (1,)), ((), ())),
                         preferred_element_type=jnp.float32) + b2_ref[...]
    lg16 = jnp.pad(lg, ((0, 0), (0, _L - NUM_E3)))
    out_ref[...] = jnp.concatenate([lg16, jax.nn.sigmoid(lg16)], axis=1)


@functools.lru_cache(maxsize=None)
def _mlp_call():
    return pl.pallas_call(
        _mlp_body,
        out_shape=jax.ShapeDtypeStruct((1, 2 * _L), jnp.float32),
    )


@functools.lru_cache(maxsize=None)
def _gather_call():
    mesh = plsc.VectorSubcoreMesh(core_axis_name="c", subcore_axis_name="s")

    @functools.partial(
        pl.kernel,
        mesh=mesh,
        out_type=[
            jax.ShapeDtypeStruct((BATCH,), jnp.float32),
            jax.ShapeDtypeStruct((BATCH,), jnp.float32),
        ],
        scratch_types=[
            pltpu.VMEM((_BPW,), jnp.int32),
            pltpu.VMEM((2 * _L,), jnp.float32),
            pltpu.VMEM((_BPW,), jnp.float32),
            pltpu.VMEM((_BPW,), jnp.float32),
            pltpu.SemaphoreType.DMA,
            pltpu.SemaphoreType.DMA,
            pltpu.SemaphoreType.DMA,
        ],
    )
    def sc_gather(idx_hbm, tlts_hbm, out_l_hbm, out_s_hbm,
                  idx_v, tlts_v, ol_v, os_v, sem_i, sem_l, sem_s):
        wid = lax.axis_index("s") * _NC + lax.axis_index("c")
        base = wid * _BPW
        idx_cp = pltpu.async_copy(idx_hbm.at[pl.ds(base, _BPW)], idx_v, sem_i)
        pltpu.sync_copy(tlts_hbm, tlts_v)
        tl = tlts_v[pl.ds(0, _L)]
        ts = tlts_v[pl.ds(_L, _L)]
        idx_cp.wait()
        for i in range(_BPW // _L):
            iv = idx_v[pl.ds(i * _L, _L)]
            ol_v[pl.ds(i * _L, _L)] = _take16(tl, iv)
        l_cp = pltpu.async_copy(ol_v, out_l_hbm.at[pl.ds(base, _BPW)], sem_l)
        for i in range(_BPW // _L):
            iv = idx_v[pl.ds(i * _L, _L)]
            os_v[pl.ds(i * _L, _L)] = _take16(ts, iv)
        s_cp = pltpu.async_copy(os_v, out_s_hbm.at[pl.ds(base, _BPW)], sem_s)
        l_cp.wait()
        s_cp.wait()

    return sc_gather


def kernel(e3_idx, table, W1, b1, W2, b2):
    idx = e3_idx.astype(jnp.int32)
    tlts = _mlp_call()(table, W1, b1.reshape(1, HID),
                       W2.reshape(1, HID), b2.reshape(1, 1))
    logits, score = _gather_call()(idx, tlts.reshape(2 * _L))
    return logits, score
